# Initial kernel scaffold; baseline (speedup 1.0000x reference)
import jax
import jax.numpy as jnp
from jax.experimental import pallas as pl


def kernel(h, pos, edge_index, t, params):
    raise NotImplementedError("write your pallas kernel here")


# speedup vs baseline: 4.5944x; 4.5944x over previous
import functools

import jax
import jax.numpy as jnp
import numpy as _np
from jax.experimental import pallas as pl

N = 10000
E = 160000
S = 256
R = 9
ED = 20
GEB = 2

BN = 400
BE = 1600


def _phi_body(s_ref, w1_ref, b1_ref, w2_ref, b2_ref, o_ref):
    x = s_ref[...]
    h1 = jax.nn.silu(x @ w1_ref[...] + b1_ref[...][None, :])
    o_ref[...] = h1 @ w2_ref[...] + b2_ref[...][None, :]


def _mlp2_pallas(x, W1, b1, W2, b2, block_rows):
    n, _ = x.shape
    d_out = W2.shape[1]
    grid = (n // block_rows,)
    return pl.pallas_call(
        _phi_body,
        grid=grid,
        in_specs=[
            pl.BlockSpec((block_rows, x.shape[1]), lambda i: (i, 0)),
            pl.BlockSpec(W1.shape, lambda i: (0, 0)),
            pl.BlockSpec(b1.shape, lambda i: (0,)),
            pl.BlockSpec(W2.shape, lambda i: (0, 0)),
            pl.BlockSpec(b2.shape, lambda i: (0,)),
        ],
        out_specs=pl.BlockSpec((block_rows, d_out), lambda i: (i, 0)),
        out_shape=jax.ShapeDtypeStruct((n, d_out), jnp.float32),
    )(x, W1, b1, W2, b2)


def _we_body(rbf_ref, wr_ref, br_ref, o_ref):
    o_ref[...] = rbf_ref[...] @ wr_ref[...] + br_ref[...][None, :]


def _we_pallas(rbf, Wr, br):
    grid = (E // BE,)
    return pl.pallas_call(
        _we_body,
        grid=grid,
        in_specs=[
            pl.BlockSpec((BE, ED), lambda i: (i, 0)),
            pl.BlockSpec(Wr.shape, lambda i: (0, 0)),
            pl.BlockSpec(br.shape, lambda i: (0,)),
        ],
        out_specs=pl.BlockSpec((BE, 3 * S), lambda i: (i, 0)),
        out_shape=jax.ShapeDtypeStruct((E, 3 * S), jnp.float32),
    )(rbf, Wr, br)


def _embed_body(h_ref, w_ref, b_ref, ctx_ref, o_ref):
    o_ref[...] = h_ref[...] @ w_ref[...] + b_ref[...][None, :] + ctx_ref[...]


def _embed_pallas(h, W, b, ctx):
    grid = (N // BN,)
    return pl.pallas_call(
        _embed_body,
        grid=grid,
        in_specs=[
            pl.BlockSpec((BN, 5), lambda i: (i, 0)),
            pl.BlockSpec(W.shape, lambda i: (0, 0)),
            pl.BlockSpec(b.shape, lambda i: (0,)),
            pl.BlockSpec((1, S), lambda i: (0, 0)),
        ],
        out_specs=pl.BlockSpec((BN, S), lambda i: (i, 0)),
        out_shape=jax.ShapeDtypeStruct((N, S), jnp.float32),
    )(h, W, b, ctx)


def _update_body(s_ref, v_ref, sagg_ref, vagg_ref, ctx_ref,
                 u_ref, vw_ref, w1_ref, b1_ref, w2_ref, b2_ref,
                 so_ref, vo_ref):
    s1 = s_ref[...] + sagg_ref[...]
    v1 = v_ref[...] + vagg_ref[...]
    U = u_ref[...]
    Vw = vw_ref[...]
    uv0 = v1[:, 0 * S:1 * S] @ U
    uv1 = v1[:, 1 * S:2 * S] @ U
    uv2 = v1[:, 2 * S:3 * S] @ U
    vv0 = v1[:, 0 * S:1 * S] @ Vw
    vv1 = v1[:, 1 * S:2 * S] @ Vw
    vv2 = v1[:, 2 * S:3 * S] @ Vw
    vn = jnp.sqrt(vv0 * vv0 + vv1 * vv1 + vv2 * vv2 + 1e-8)
    pre = s1 @ w1_ref[0:S, :] + vn @ w1_ref[S:2 * S, :] + b1_ref[...][None, :]
    a = jax.nn.silu(pre) @ w2_ref[...] + b2_ref[...][None, :]
    dot = uv0 * vv0 + uv1 * vv1 + uv2 * vv2
    a_vv = a[:, 2 * S:3 * S]
    so_ref[...] = (s1 + a[:, 0:S] + a[:, S:2 * S] * dot + ctx_ref[...])
    vo_ref[...] = v1 + jnp.concatenate(
        [a_vv * uv0, a_vv * uv1, a_vv * uv2], axis=1)


def _update_pallas(s, v_cat, s_agg, v_agg, ctx, U, Vw, W1, b1, W2, b2):
    grid = (N // BN,)
    return pl.pallas_call(
        _update_body,
        grid=grid,
        in_specs=[
            pl.BlockSpec((BN, S), lambda i: (i, 0)),
            pl.BlockSpec((BN, 3 * S), lambda i: (i, 0)),
            pl.BlockSpec((BN, S), lambda i: (i, 0)),
            pl.BlockSpec((BN, 3 * S), lambda i: (i, 0)),
            pl.BlockSpec((1, S), lambda i: (0, 0)),
            pl.BlockSpec(U.shape, lambda i: (0, 0)),
            pl.BlockSpec(Vw.shape, lambda i: (0, 0)),
            pl.BlockSpec(W1.shape, lambda i: (0, 0)),
            pl.BlockSpec(b1.shape, lambda i: (0,)),
            pl.BlockSpec(W2.shape, lambda i: (0, 0)),
            pl.BlockSpec(b2.shape, lambda i: (0,)),
        ],
        out_specs=[
            pl.BlockSpec((BN, S), lambda i: (i, 0)),
            pl.BlockSpec((BN, 3 * S), lambda i: (i, 0)),
        ],
        out_shape=[
            jax.ShapeDtypeStruct((N, S), jnp.float32),
            jax.ShapeDtypeStruct((N, 3 * S), jnp.float32),
        ],
    )(s, v_cat, s_agg, v_agg, ctx, U, Vw, W1, b1, W2, b2)


def _geb_body(s_ref, v_ref, ctx_ref, wv1_ref, wv2_ref,
              w1_ref, b1_ref, w2_ref, b2_ref, so_ref, vo_ref):
    v = v_ref[...]
    Wv1 = wv1_ref[...]
    Wv2 = wv2_ref[...]
    v10 = v[:, 0 * S:1 * S] @ Wv1
    v11 = v[:, 1 * S:2 * S] @ Wv1
    v12 = v[:, 2 * S:3 * S] @ Wv1
    v20 = v[:, 0 * S:1 * S] @ Wv2
    v21 = v[:, 1 * S:2 * S] @ Wv2
    v22 = v[:, 2 * S:3 * S] @ Wv2
    n2 = jnp.sqrt(v20 * v20 + v21 * v21 + v22 * v22 + 1e-8)
    pre = s_ref[...] @ w1_ref[0:S, :] + n2 @ w1_ref[S:2 * S, :] + b1_ref[...][None, :]
    xg = jax.nn.silu(pre) @ w2_ref[...] + b2_ref[...][None, :]
    gate = xg[:, S:2 * S]
    so_ref[...] = xg[:, 0:S] + ctx_ref[...]
    vo_ref[...] = jnp.concatenate([gate * v10, gate * v11, gate * v12], axis=1)


def _geb_pallas(s, v_cat, ctx, Wv1, Wv2, W1, b1, W2, b2):
    grid = (N // BN,)
    return pl.pallas_call(
        _geb_body,
        grid=grid,
        in_specs=[
            pl.BlockSpec((BN, S), lambda i: (i, 0)),
            pl.BlockSpec((BN, 3 * S), lambda i: (i, 0)),
            pl.BlockSpec((1, S), lambda i: (0, 0)),
            pl.BlockSpec(Wv1.shape, lambda i: (0, 0)),
            pl.BlockSpec(Wv2.shape, lambda i: (0, 0)),
            pl.BlockSpec(W1.shape, lambda i: (0, 0)),
            pl.BlockSpec(b1.shape, lambda i: (0,)),
            pl.BlockSpec(W2.shape, lambda i: (0, 0)),
            pl.BlockSpec(b2.shape, lambda i: (0,)),
        ],
        out_specs=[
            pl.BlockSpec((BN, S), lambda i: (i, 0)),
            pl.BlockSpec((BN, 3 * S), lambda i: (i, 0)),
        ],
        out_shape=[
            jax.ShapeDtypeStruct((N, S), jnp.float32),
            jax.ShapeDtypeStruct((N, 3 * S), jnp.float32),
        ],
    )(s, v_cat, ctx, Wv1, Wv2, W1, b1, W2, b2)


def _readout_body(s_ref, v_ref, iw1_ref, ib1_ref, iw2_ref, ib2_ref,
                  ew1_ref, eb1_ref, ew2_ref, eb2_ref, wvec_ref,
                  eo_ref, io_ref):
    s = s_ref[...]
    v = v_ref[...]
    inv = jax.nn.silu(s @ iw1_ref[...] + ib1_ref[...][None, :]) @ iw2_ref[...] \
        + ib2_ref[...][None, :]
    gate = jax.nn.silu(s @ ew1_ref[...] + eb1_ref[...][None, :]) @ ew2_ref[...] \
        + eb2_ref[...][None, :]
    wv = wvec_ref[...]
    vec0 = jnp.sum(v[:, 0 * S:1 * S] * wv, axis=1, keepdims=True)
    vec1 = jnp.sum(v[:, 1 * S:2 * S] * wv, axis=1, keepdims=True)
    vec2 = jnp.sum(v[:, 2 * S:3 * S] * wv, axis=1, keepdims=True)
    eo_ref[...] = gate * jnp.concatenate([vec0, vec1, vec2], axis=1)
    io_ref[...] = inv


def _readout_pallas(s, v_cat, p):
    grid = (N // BN,)
    wvec = p['equi_wvec'][None, :]
    return pl.pallas_call(
        _readout_body,
        grid=grid,
        in_specs=[
            pl.BlockSpec((BN, S), lambda i: (i, 0)),
            pl.BlockSpec((BN, 3 * S), lambda i: (i, 0)),
            pl.BlockSpec(p['inv_W1'].shape, lambda i: (0, 0)),
            pl.BlockSpec(p['inv_b1'].shape, lambda i: (0,)),
            pl.BlockSpec(p['inv_W2'].shape, lambda i: (0, 0)),
            pl.BlockSpec(p['inv_b2'].shape, lambda i: (0,)),
            pl.BlockSpec(p['equi_W1'].shape, lambda i: (0, 0)),
            pl.BlockSpec(p['equi_b1'].shape, lambda i: (0,)),
            pl.BlockSpec(p['equi_W2'].shape, lambda i: (0, 0)),
            pl.BlockSpec(p['equi_b2'].shape, lambda i: (0,)),
            pl.BlockSpec((1, S), lambda i: (0, 0)),
        ],
        out_specs=[
            pl.BlockSpec((BN, 3), lambda i: (i, 0)),
            pl.BlockSpec((BN, 5), lambda i: (i, 0)),
        ],
        out_shape=[
            jax.ShapeDtypeStruct((N, 3), jnp.float32),
            jax.ShapeDtypeStruct((N, 5), jnp.float32),
        ],
    )(s, v_cat, p['inv_W1'], p['inv_b1'], p['inv_W2'], p['inv_b2'],
      p['equi_W1'], p['equi_b1'], p['equi_W2'], p['equi_b2'], wvec)


def _rbf_body(rij_ref, d_ref, dirv_ref, rbf_ref):
    rij = rij_ref[...]
    d2 = jnp.sum(rij * rij, axis=1, keepdims=True)
    d = jnp.maximum(jnp.sqrt(d2), 1e-6)
    dirv_ref[...] = rij / d
    ci = jax.lax.broadcasted_iota(jnp.int32, (1, ED), 1)
    centers = ci.astype(jnp.float32) * (5.0 / (ED - 1))
    d_ref[...] = d
    rbf_ref[...] = jnp.exp(-10.0 * (d - centers) ** 2)


def _rbf_pallas(r_ij):
    grid = (E // BE,)
    return pl.pallas_call(
        _rbf_body,
        grid=grid,
        in_specs=[pl.BlockSpec((BE, 3), lambda i: (i, 0))],
        out_specs=[
            pl.BlockSpec((BE, 1), lambda i: (i, 0)),
            pl.BlockSpec((BE, 3), lambda i: (i, 0)),
            pl.BlockSpec((BE, ED), lambda i: (i, 0)),
        ],
        out_shape=[
            jax.ShapeDtypeStruct((E, 1), jnp.float32),
            jax.ShapeDtypeStruct((E, 3), jnp.float32),
            jax.ShapeDtypeStruct((E, ED), jnp.float32),
        ],
    )(r_ij)


def kernel(h, pos, edge_index, t, params):
    p = params
    row = edge_index[0]
    col = edge_index[1]

    r_ij = pos[col] - pos[row]
    _, dirv, rbf = _rbf_pallas(r_ij)

    half = S // 2
    freqs = jnp.exp(-jnp.log(10000.0) * jnp.arange(half) / half)
    ang = t[:, None] * freqs[None, :]
    ctx = _mlp2_ref(jnp.concatenate([jnp.sin(ang), jnp.cos(ang)], axis=1),
                    p['time_W1'], p['time_b1'], p['time_W2'], p['time_b2'])

    s = _embed_pallas(h, p['emb_W'], p['emb_b'], ctx)
    v_cat = jnp.zeros((N, 3 * S), dtype=jnp.float32)

    for r in range(R):
        phi = _mlp2_pallas(s, p['msg_W1'][r], p['msg_b1'][r],
                           p['msg_W2'][r], p['msg_b2'][r], BN)
        We = _we_pallas(rbf, p['msg_Wr'][r], p['msg_br'][r])
        x = phi[row] * We
        ds = x[:, 0:S]
        dvv = x[:, S:2 * S]
        dvs = x[:, 2 * S:]
        v_row = v_cat[row]
        dvm = jnp.concatenate([
            dvv * v_row[:, 0 * S:1 * S] + dvs * dirv[:, 0:1],
            dvv * v_row[:, 1 * S:2 * S] + dvs * dirv[:, 1:2],
            dvv * v_row[:, 2 * S:3 * S] + dvs * dirv[:, 2:3],
        ], axis=1)
        s_agg = jax.ops.segment_sum(ds, col, num_segments=N)
        v_agg = jax.ops.segment_sum(dvm, col, num_segments=N)
        s, v_cat = _update_pallas(s, v_cat, s_agg, v_agg, ctx,
                                  p['upd_U'][r], p['upd_V'][r],
                                  p['upd_W1'][r], p['upd_b1'][r],
                                  p['upd_W2'][r], p['upd_b2'][r])

    for g in range(GEB):
        s, v_cat = _geb_pallas(s, v_cat, ctx,
                               p['geb_Wv1'][g], p['geb_Wv2'][g],
                               p['geb_W1'][g], p['geb_b1'][g],
                               p['geb_W2'][g], p['geb_b2'][g])

    equi_out, inv_out = _readout_pallas(s, v_cat, p)
    return (equi_out, inv_out)


def _mlp2_ref(x, W1, b1, W2, b2):
    return jax.nn.silu(x @ W1 + b1) @ W2 + b2
